# baseline (speedup 1.0000x reference)
import functools

import jax
import jax.numpy as jnp
from jax import lax
from jax.experimental import pallas as pl
from jax.experimental.pallas import tpu as pltpu
from jax.experimental.pallas import tpu_sc as plsc

_L = 16
_NC = 2
_NS = 16
_NW = _NC * _NS
_CHUNK = 16384
_UNROLL = 4


def _gal_sc_body(x_hbm, par_hbm, out_hbm, par_v, inb, outb,
                 in_sem0, in_sem1, out_sem0, out_sem1):
    total = x_hbm.shape[0]
    n_chunks = total // (_NW * _CHUNK)
    wid = lax.axis_index("s") * _NC + lax.axis_index("c")
    base = wid * (n_chunks * _CHUNK)

    pltpu.sync_copy(par_hbm, par_v)
    kr = [par_v[j] for j in range(5)]
    kl = [par_v[5 + j] for j in range(5)]
    bg = par_v[10]
    rj = [par_v[11 + j] for j in range(4)]
    cr = [kr[0]] + [kr[j] - kr[j - 1] for j in range(1, 5)]
    cl = [-kl[0]] + [kl[j - 1] - kl[j] for j in range(1, 5)]

    in_sems = (in_sem0, in_sem1)
    out_sems = (out_sem0, out_sem1)

    def in_copy(g, b):
        return pltpu.make_async_copy(
            x_hbm.at[pl.ds(base + g * _CHUNK, _CHUNK)], inb.at[b], in_sems[b])

    def out_copy(g, b):
        return pltpu.make_async_copy(
            outb.at[b], out_hbm.at[pl.ds(base + g * _CHUNK, _CHUNK)],
            out_sems[b])

    def compute(b):
        def body(i, carry):
            off = i * (_L * _UNROLL)
            for u in range(_UNROLL):
                sl = pl.ds(off + u * _L, _L)
                xv = inb[b, sl]
                a = jnp.abs(xv)
                pos = xv > 0.0
                acc = jnp.where(pos, cr[0], cl[0]) * a + bg
                for j in range(1, 5):
                    r = jnp.maximum(a - rj[j - 1], 0.0)
                    acc = acc + jnp.where(pos, cr[j], cl[j]) * r
                outb[b, sl] = acc
            return carry
        lax.fori_loop(0, _CHUNK // (_L * _UNROLL), body, 0)

    in_copy(0, 0).start()
    in_copy(1, 1).start()

    def outer(t, carry):
        for b in range(2):
            g = t * 2 + b
            in_copy(g, b).wait()

            @pl.when(g >= 2)
            def _():
                out_copy(g - 2, b).wait()

            compute(b)
            out_copy(g, b).start()

            @pl.when(g + 2 < n_chunks)
            def _():
                in_copy(g + 2, b).start()
        return carry

    lax.fori_loop(0, n_chunks // 2, outer, 0)
    out_copy(n_chunks - 2, 0).wait()
    out_copy(n_chunks - 1, 1).wait()


def _gal_sc(x_flat, par):
    total = x_flat.shape[0]
    assert total % (_NW * _CHUNK * 2) == 0
    fn = functools.partial(
        pl.kernel,
        mesh=plsc.VectorSubcoreMesh(core_axis_name="c", subcore_axis_name="s"),
        out_type=jax.ShapeDtypeStruct((total,), jnp.float32),
        scratch_types=[
            pltpu.VMEM((16, _L), jnp.float32),
            pltpu.VMEM((2, _CHUNK), jnp.float32),
            pltpu.VMEM((2, _CHUNK), jnp.float32),
            pltpu.SemaphoreType.DMA,
            pltpu.SemaphoreType.DMA,
            pltpu.SemaphoreType.DMA,
            pltpu.SemaphoreType.DMA,
        ],
    )(_gal_sc_body)
    return fn(x_flat, par)


def kernel(x, p_l, p_r, k_l, k_r, b_g):
    del p_l
    orig_shape = x.shape
    par = jnp.concatenate([
        k_r.reshape(-1), k_l.reshape(-1), b_g.reshape(-1),
        p_r.reshape(-1)[1:], jnp.zeros((1,), jnp.float32),
    ])
    par = jnp.broadcast_to(par.reshape(16, 1), (16, _L))
    out = _gal_sc(x.reshape(-1), par)
    return out.reshape(orig_shape)


# speedup vs baseline: 2.2592x; 2.2592x over previous
import functools

import jax
import jax.numpy as jnp
from jax import lax
from jax.experimental import pallas as pl
from jax.experimental.pallas import tpu as pltpu
from jax.experimental.pallas import tpu_sc as plsc

_L = 16
_NC = 2
_NS = 16
_NW = _NC * _NS
_CHUNK = 16384
_UNROLL = 4

_UNIT = _NW * _CHUNK
_COLS = 2048

_SC_UNITS_OF_64 = 10


def _gal_sc_body(x_hbm, par_hbm, out_hbm, par_v, inb, outb,
                 in_sem0, in_sem1, out_sem0, out_sem1):
    total = x_hbm.shape[0]
    n_chunks = total // (_NW * _CHUNK)
    wid = lax.axis_index("s") * _NC + lax.axis_index("c")
    base = wid * (n_chunks * _CHUNK)

    pltpu.sync_copy(par_hbm, par_v)
    kr = [par_v[j] for j in range(5)]
    kl = [par_v[5 + j] for j in range(5)]
    bg = par_v[10]
    rj = [par_v[11 + j] for j in range(4)]
    cr = [kr[0]] + [kr[j] - kr[j - 1] for j in range(1, 5)]
    cl = [-kl[0]] + [kl[j - 1] - kl[j] for j in range(1, 5)]

    in_sems = (in_sem0, in_sem1)
    out_sems = (out_sem0, out_sem1)

    def in_copy(g, b):
        return pltpu.make_async_copy(
            x_hbm.at[pl.ds(base + g * _CHUNK, _CHUNK)], inb.at[b], in_sems[b])

    def out_copy(g, b):
        return pltpu.make_async_copy(
            outb.at[b], out_hbm.at[pl.ds(base + g * _CHUNK, _CHUNK)],
            out_sems[b])

    def compute(b):
        def body(i, carry):
            off = i * (_L * _UNROLL)
            for u in range(_UNROLL):
                sl = pl.ds(off + u * _L, _L)
                xv = inb[b, sl]
                a = jnp.abs(xv)
                pos = xv > 0.0
                acc = jnp.where(pos, cr[0], cl[0]) * a + bg
                for j in range(1, 5):
                    r = jnp.maximum(a - rj[j - 1], 0.0)
                    acc = acc + jnp.where(pos, cr[j], cl[j]) * r
                outb[b, sl] = acc
            return carry
        lax.fori_loop(0, _CHUNK // (_L * _UNROLL), body, 0)

    in_copy(0, 0).start()
    in_copy(1, 1).start()

    def outer(t, carry):
        for b in range(2):
            g = t * 2 + b
            in_copy(g, b).wait()

            @pl.when(g >= 2)
            def _():
                out_copy(g - 2, b).wait()

            compute(b)
            out_copy(g, b).start()

            @pl.when(g + 2 < n_chunks)
            def _():
                in_copy(g + 2, b).start()
        return carry

    lax.fori_loop(0, n_chunks // 2, outer, 0)
    out_copy(n_chunks - 2, 0).wait()
    out_copy(n_chunks - 1, 1).wait()


def _gal_sc(x_flat, par):
    total = x_flat.shape[0]
    assert total % (_NW * _CHUNK * 2) == 0
    fn = functools.partial(
        pl.kernel,
        mesh=plsc.VectorSubcoreMesh(core_axis_name="c", subcore_axis_name="s"),
        out_type=jax.ShapeDtypeStruct((total,), jnp.float32),
        scratch_types=[
            pltpu.VMEM((16, _L), jnp.float32),
            pltpu.VMEM((2, _CHUNK), jnp.float32),
            pltpu.VMEM((2, _CHUNK), jnp.float32),
            pltpu.SemaphoreType.DMA,
            pltpu.SemaphoreType.DMA,
            pltpu.SemaphoreType.DMA,
            pltpu.SemaphoreType.DMA,
        ],
    )(_gal_sc_body)
    return fn(x_flat, par)


def _gal_tc_body(p_l, p_r, k_l, k_r, b_g, x_ref, o_ref):
    x = x_ref[...]
    acc = (
        jnp.maximum(x, 0.0) * k_r[0, 0]
        + jnp.minimum(x, 0.0) * k_l[0, 0]
        + b_g[0]
    )
    for j in range(1, 5):
        acc += (k_r[j, 0] - k_r[j - 1, 0]) * jnp.maximum(x - p_r[j, 0], 0.0)
        acc += (k_l[j, 0] - k_l[j - 1, 0]) * jnp.minimum(x - p_l[j, 0], 0.0)
    o_ref[...] = acc


def _gal_tc(x2, p_l, p_r, k_l, k_r, b_g, block_rows):
    rows, cols = x2.shape
    assert rows % block_rows == 0
    smem = pl.BlockSpec(memory_space=pltpu.SMEM)
    return pl.pallas_call(
        _gal_tc_body,
        grid=(rows // block_rows,),
        in_specs=[
            smem, smem, smem, smem, smem,
            pl.BlockSpec((block_rows, cols), lambda i: (i, 0)),
        ],
        out_specs=pl.BlockSpec((block_rows, cols), lambda i: (i, 0)),
        out_shape=jax.ShapeDtypeStruct((rows, cols), x2.dtype),
    )(p_l, p_r, k_l, k_r, b_g, x2)


def kernel(x, p_l, p_r, k_l, k_r, b_g):
    orig_shape = x.shape
    x2 = x.reshape(-1, _COLS)
    rows = x2.shape[0]
    n_units = rows * _COLS // _UNIT
    sc_units = _SC_UNITS_OF_64
    sc_rows = sc_units * (_UNIT // _COLS)
    tc_rows = rows - sc_rows

    par = jnp.concatenate([
        k_r.reshape(-1), k_l.reshape(-1), b_g.reshape(-1),
        p_r.reshape(-1)[1:], jnp.zeros((1,), jnp.float32),
    ])
    par = jnp.broadcast_to(par.reshape(16, 1), (16, _L))

    out_tc = _gal_tc(x2[:tc_rows], p_l, p_r, k_l, k_r, b_g, block_rows=256)
    out_sc = _gal_sc(x2[tc_rows:].reshape(-1), par).reshape(sc_rows, _COLS)
    return jnp.concatenate([out_tc, out_sc], axis=0).reshape(orig_shape)
